# baseline (speedup 1.0000x reference)
import functools
import jax
import jax.numpy as jnp
from jax import lax
from jax.experimental import pallas as pl
from jax.experimental.pallas import tpu as pltpu
from jax.experimental.pallas import tpu_sc as plsc

B = 1024
K = 100000
CTX = 64
COARSE = 80
TOPM = 20
PRED = 96
CH = 7
TEMP = 0.1
ALPHA = 0.7
GH = 128

KP = 100352
CHW = 128
NCH = KP // CHW
QB_A = 256
QB_B = 256
KB_B = 2048
QB_D = 128
SER = PRED * CH
NEG = -1e30


def _gelu(x):
    return 0.5 * x * (1.0 + lax.erf(x * 0.7071067811865476))


def _bdot(a, b):
    return jnp.dot(a.astype(jnp.bfloat16), b.astype(jnp.bfloat16),
                   preferred_element_type=jnp.float32)


def _b(x):
    return x.astype(jnp.bfloat16).astype(jnp.float32)


def _ctx_body(cat_ref, exo_ref, ls_ref,
              cw1_ref, cb1_ref, cw2_ref, cb2_ref,
              ew1_ref, eb1_ref, ew2_ref, eb2_ref,
              lw1_ref, lb1_ref, lw2_ref, lb2_ref,
              g_ref, b_ref, qn_ref):
    cat = cat_ref[...]
    cc = _bdot(_gelu(_bdot(cat, cw1_ref[...]) + cb1_ref[...]),
               cw2_ref[...]) + cb2_ref[...]
    exo = _b(exo_ref[...])
    ew1 = _b(ew1_ref[...])
    eh = eb1_ref[...] + jnp.zeros((QB_A, CTX), jnp.float32)
    for j in range(8):
        eh = eh + exo[:, j:j + 1] * ew1[j:j + 1, :]
    eo = _bdot(_gelu(eh), ew2_ref[...]) + eb2_ref[...]
    cc = cc + eo
    ls = _b(ls_ref[...])
    lw1 = _b(lw1_ref[...])
    acc = jnp.zeros((QB_A, CTX), jnp.float32)
    for p in range(3):
        lh = lb1_ref[...] + jnp.zeros((QB_A, CTX), jnp.float32)
        for j in range(4):
            lh = lh + ls[:, 4 * p + j:4 * p + j + 1] * lw1[j:j + 1, :]
        lo = _bdot(_gelu(lh), lw2_ref[...]) + lb2_ref[...]
        h = cc + lo
        mu = jnp.mean(h, axis=1, keepdims=True)
        var = jnp.mean((h - mu) ** 2, axis=1, keepdims=True)
        acc = acc + ((h - mu) / jnp.sqrt(var + 1e-5) * g_ref[...] + b_ref[...])
    q = acc / 3.0
    qn = q / (jnp.sqrt(jnp.sum(q * q, axis=1, keepdims=True)) + 1e-8)
    qn_ref[...] = qn


def _ctx_call(cat, exo, ls12, cw1, cb1, cw2, cb2, ew1, eb1, ew2, eb2,
              lw1, lb1, lw2, lb2, g, b):
    nq = B // QB_A
    full = lambda shp: pl.BlockSpec(shp, lambda i: (0, 0))
    return pl.pallas_call(
        _ctx_body,
        grid=(nq,),
        in_specs=[
            pl.BlockSpec((QB_A, 272), lambda i: (i, 0)),
            pl.BlockSpec((QB_A, 8), lambda i: (i, 0)),
            pl.BlockSpec((QB_A, 12), lambda i: (i, 0)),
            full((272, CTX)), full((1, CTX)), full((CTX, CTX)), full((1, CTX)),
            full((8, CTX)), full((1, CTX)), full((CTX, CTX)), full((1, CTX)),
            full((4, CTX)), full((1, CTX)), full((CTX, CTX)), full((1, CTX)),
            full((1, CTX)), full((1, CTX)),
        ],
        out_specs=pl.BlockSpec((QB_A, CTX), lambda i: (i, 0)),
        out_shape=jax.ShapeDtypeStruct((B, CTX), jnp.float32),
    )(cat, exo, ls12, cw1, cb1, cw2, cb2, ew1, eb1, ew2, eb2,
      lw1, lb1, lw2, lb2, g, b)


def _sim_body(qn_ref, keys_ref, sim_ref, m_ref):
    k = pl.program_id(0)
    keys = keys_ref[...][:, :CTX]
    nrm = jnp.sqrt(jnp.sum(keys * keys, axis=1, keepdims=True))
    kn = keys / (nrm + 1e-8)
    sim = lax.dot_general(qn_ref[...].astype(jnp.bfloat16),
                          kn.astype(jnp.bfloat16), (((1,), (1,)), ((), ())),
                          preferred_element_type=jnp.float32)
    col = k * KB_B + lax.broadcasted_iota(jnp.int32, (QB_B, KB_B), 1)
    sim = jnp.where(col < K, sim, NEG)
    sim3 = sim.reshape(QB_B, KB_B // CHW, CHW)
    sim_ref[...] = sim3
    m_ref[...] = jnp.max(sim3, axis=2).reshape(1, QB_B, KB_B // CHW)


def _sim_call(qn, keys_pad):
    nk = KP // KB_B
    nq = B // QB_B
    return pl.pallas_call(
        _sim_body,
        grid=(nk, nq),
        in_specs=[
            pl.BlockSpec((QB_B, CTX), lambda k, q: (q, 0)),
            pl.BlockSpec((KB_B, 128), lambda k, q: (k, 0)),
        ],
        out_specs=[
            pl.BlockSpec((QB_B, KB_B // CHW, CHW), lambda k, q: (q, k, 0)),
            pl.BlockSpec((1, QB_B, KB_B // CHW), lambda k, q: (k, q, 0)),
        ],
        out_shape=[
            jax.ShapeDtypeStruct((B, NCH, CHW), jnp.float32),
            jax.ShapeDtypeStruct((KP // KB_B, B, KB_B // CHW), jnp.float32),
        ],
    )(qn, keys_pad)


def _splat_f(x):
    return lax.broadcast(x, (16,))


def _topk_body(m_hbm, sim_hbm, keys_hbm, cv_hbm, ci_hbm, ck_hbm, *bufs):
    b0 = bufs[0:10]
    b1 = bufs[10:20]
    sem0, sem1 = bufs[20], bufs[21]
    wid = lax.axis_index("s") * 2 + lax.axis_index("c")
    nper = B // 32
    iota = lax.iota(jnp.int32, 16)
    lane0 = iota == 0
    negv = jnp.full((16,), NEG, jnp.float32)
    big = jnp.int32(1 << 20)

    for m_v, gm_v, cidx_v, absidx_v, cm_v, ch_v, cv_v, ci_v, ci80_v, ck_v \
            in (b0, b1):
        for i in range(8):
            cv_v[pl.ds(16 * i, 16)] = negv
            ci_v[pl.ds(16 * i, 16)] = jnp.zeros((16,), jnp.int32)
        cm_v[pl.ds(80, 16)] = negv
        gm_v[pl.ds(48, 16)] = negv

    def per_pair(pi, _):
        q0 = wid * nper + 2 * pi
        qs = (q0, q0 + 1)
        d0 = pltpu.async_copy(m_hbm.at[:, qs[0]], b0[0], sem0)
        d1 = pltpu.async_copy(m_hbm.at[:, qs[1]], b1[0], sem1)
        d0.wait()
        d1.wait()
        for j in range(NCH // 16):
            jv = lax.broadcast(jnp.int32(j), (16,))
            for (m_v, gm_v, *_r) in (b0, b1):
                plsc.store_scatter(gm_v, [jv], _splat_f(jnp.max(m_v[j, :])),
                                   mask=lane0)

        def sel_body(t, _c):
            tv = lax.broadcast(t, (16,))
            for (m_v, gm_v, cidx_v, _a, cm_v, *_r) in (b0, b1):
                g = jnp.full((16,), NEG, jnp.float32)
                for i in range(4):
                    g = jnp.maximum(g, gm_v[pl.ds(16 * i, 16)])
                s = jnp.max(g)
                jst = big
                for i in range(4):
                    v = gm_v[pl.ds(16 * i, 16)]
                    jst = jnp.minimum(jst, jnp.min(
                        jnp.where(v == s, iota + 16 * i, big)))
                jv = lax.broadcast(jst, (16,))
                mrow = plsc.load_gather(m_v, [jv, iota])
                lst = jnp.min(jnp.where(mrow == s, iota, big))
                cst = jst * 16 + lst
                plsc.store_scatter(cidx_v, [tv], lax.broadcast(cst, (16,)),
                                   mask=lane0)
                plsc.store_scatter(cm_v, [tv], _splat_f(s), mask=lane0)
                plsc.store_scatter(m_v, [jv, lax.broadcast(lst, (16,))],
                                   negv, mask=lane0)
                mrow2 = plsc.load_gather(m_v, [jv, iota])
                plsc.store_scatter(gm_v, [jv], _splat_f(jnp.max(mrow2)),
                                   mask=lane0)
            return 0

        lax.fori_loop(0, COARSE, sel_body, 0)

        for i in range(5):
            b0[3][pl.ds(16 * i, 16)] = b0[2][pl.ds(16 * i, 16)] + qs[0] * NCH
            b1[3][pl.ds(16 * i, 16)] = b1[2][pl.ds(16 * i, 16)] + qs[1] * NCH
        g0 = pltpu.async_copy(sim_hbm.at[b0[3]], b0[5], sem0)
        g1 = pltpu.async_copy(sim_hbm.at[b1[3]], b1[5], sem1)
        g0.wait()
        g1.wait()

        def ext_body(t, _c):
            tv = lax.broadcast(t, (16,))
            for (_m, _g, cidx_v, _a, cm_v, ch_v, cv_v, ci_v, ci80_v, _ck) \
                    in (b0, b1):
                s = jnp.float32(NEG)
                for i in range(6):
                    s = jnp.maximum(s, jnp.max(cm_v[pl.ds(16 * i, 16)]))
                pst = big
                for i in range(6):
                    v = cm_v[pl.ds(16 * i, 16)]
                    pst = jnp.minimum(pst, jnp.min(
                        jnp.where(v == s, iota + 16 * i, big)))
                pv = lax.broadcast(pst, (16,))
                lst = big
                for kk in range(8):
                    v = plsc.load_gather(ch_v, [pv, 16 * kk + iota])
                    lst = jnp.minimum(lst, jnp.min(
                        jnp.where(v == s, 16 * kk + iota, big)))
                cch = jnp.max(plsc.load_gather(cidx_v, [pv]))
                gi = cch * CHW + lst
                plsc.store_scatter(cv_v, [tv], _splat_f(s), mask=lane0)
                plsc.store_scatter(ci_v, [tv], lax.broadcast(gi, (16,)),
                                   mask=lane0)
                plsc.store_scatter(ci80_v, [tv], lax.broadcast(gi, (16,)),
                                   mask=lane0)
                plsc.store_scatter(ch_v, [pv, lax.broadcast(lst, (16,))],
                                   negv, mask=lane0)
                nm = jnp.full((16,), NEG, jnp.float32)
                for kk in range(8):
                    nm = jnp.maximum(nm, plsc.load_gather(
                        ch_v, [pv, 16 * kk + iota]))
                plsc.store_scatter(cm_v, [pv], _splat_f(jnp.max(nm)),
                                   mask=lane0)
            return 0

        lax.fori_loop(0, COARSE, ext_body, 0)

        k0 = pltpu.async_copy(keys_hbm.at[b0[8]], b0[9], sem0)
        k1 = pltpu.async_copy(keys_hbm.at[b1[8]], b1[9], sem1)
        k0.wait()
        k1.wait()
        for bb, q in ((b0, qs[0]), (b1, qs[1])):
            pltpu.sync_copy(bb[6], cv_hbm.at[q])
            pltpu.sync_copy(bb[7], ci_hbm.at[q])
            pltpu.sync_copy(bb[9], ck_hbm.at[q])
        return 0

    lax.fori_loop(0, nper // 2, per_pair, 0)


def _topk_call(m, sim2, keys_pad):
    mesh = plsc.VectorSubcoreMesh(core_axis_name="c", subcore_axis_name="s")
    f = pl.kernel(
        _topk_body,
        out_type=[
            jax.ShapeDtypeStruct((B, 128), jnp.float32),
            jax.ShapeDtypeStruct((B, 128), jnp.int32),
            jax.ShapeDtypeStruct((B, COARSE, 128), jnp.float32),
        ],
        mesh=mesh,
        compiler_params=pltpu.CompilerParams(needs_layout_passes=False),
        scratch_types=[
            pltpu.VMEM((NCH // 16, 16), jnp.float32),
            pltpu.VMEM((64,), jnp.float32),
            pltpu.VMEM((COARSE,), jnp.int32),
            pltpu.VMEM((COARSE,), jnp.int32),
            pltpu.VMEM((96,), jnp.float32),
            pltpu.VMEM((COARSE, CHW), jnp.float32),
            pltpu.VMEM((128,), jnp.float32),
            pltpu.VMEM((128,), jnp.int32),
            pltpu.VMEM((COARSE,), jnp.int32),
            pltpu.VMEM((COARSE, 128), jnp.float32),
        ] * 2 + [
            pltpu.SemaphoreType.DMA,
            pltpu.SemaphoreType.DMA,
        ],
    )
    return f(m, sim2, keys_pad)


def _gate_body(qn_ref, ck_ref, cv_ref, cif_ref,
               wq_ref, wk_ref, wcv_ref, wrk_ref, b1_ref, w2_ref, b2_ref,
               wf_ref, fif_ref):
    ckf = ck_ref[...][:, :CTX]
    nrm = jnp.sqrt(jnp.sum(ckf * ckf, axis=1, keepdims=True))
    ckn = ckf / (nrm + 1e-8)
    hk = _bdot(ckn, wk_ref[...])
    hq = _bdot(qn_ref[...], wq_ref[...])
    h3 = hk.reshape(QB_D, COARSE, GH) + hq[:, None, :]
    cv = cv_ref[...][:, :COARSE]
    h3 = h3 + _b(cv)[..., None] * _b(wcv_ref[...])[None, :, :]
    rank = lax.broadcasted_iota(jnp.int32, (QB_D, COARSE, 1), 1).astype(
        jnp.float32) / COARSE
    h3 = h3 + _b(rank) * _b(wrk_ref[...])[None, :, :]
    h3 = h3 + b1_ref[...][None, :, :]
    g3 = _gelu(h3)
    gl = jnp.sum(_b(g3) * _b(w2_ref[...])[None, :, :], axis=2) + b2_ref[0, 0]
    gate = jax.nn.sigmoid(gl)
    fused = ALPHA * cv + (1.0 - ALPHA) * gate
    cif = cif_ref[...][:, :COARSE]
    lane = lax.broadcasted_iota(jnp.int32, (QB_D, COARSE), 1)
    cur = fused
    fvals = []
    fidx = []
    for t in range(TOPM):
        a = jnp.argmax(cur, axis=1)
        oh = lane == a[:, None]
        fvals.append(jnp.max(cur, axis=1))
        fidx.append(jnp.sum(jnp.where(oh, cif, 0.0), axis=1))
        cur = jnp.where(oh, NEG, cur)
    fv = jnp.stack(fvals, axis=1)
    fi = jnp.stack(fidx, axis=1)
    e = jnp.exp((fv - fv[:, :1]) / TEMP)
    w = e / jnp.sum(e, axis=1, keepdims=True)
    wf_ref[...] = jnp.concatenate(
        [w, jnp.zeros((QB_D, 128 - TOPM), jnp.float32)], axis=1)
    fif_ref[...] = jnp.concatenate(
        [fi, jnp.broadcast_to(fi[:, :1], (QB_D, 128 - TOPM))], axis=1)


def _gate_call(qn, ckflat, cv, cif, wq, wk, wcv, wrk, b1, w2t, b2):
    nq = B // QB_D
    full = lambda shp: pl.BlockSpec(shp, lambda i: (0, 0))
    return pl.pallas_call(
        _gate_body,
        grid=(nq,),
        in_specs=[
            pl.BlockSpec((QB_D, CTX), lambda i: (i, 0)),
            pl.BlockSpec((QB_D * COARSE, 128), lambda i: (i, 0)),
            pl.BlockSpec((QB_D, 128), lambda i: (i, 0)),
            pl.BlockSpec((QB_D, 128), lambda i: (i, 0)),
            full((CTX, GH)), full((CTX, GH)), full((1, GH)), full((1, GH)),
            full((1, GH)), full((1, GH)), full((1, 1)),
        ],
        out_specs=[
            pl.BlockSpec((QB_D, 128), lambda i: (i, 0)),
            pl.BlockSpec((QB_D, 128), lambda i: (i, 0)),
        ],
        out_shape=[
            jax.ShapeDtypeStruct((B, 128), jnp.float32),
            jax.ShapeDtypeStruct((B, 128), jnp.float32),
        ],
    )(qn, ckflat, cv, cif, wq, wk, wcv, wrk, b1, w2t, b2)


def _series_body(series_hbm, fi_hbm, w_hbm, out_hbm,
                 fi_v, w_v, sv, ob, sem):
    wid = lax.axis_index("s") * 2 + lax.axis_index("c")
    nper = B // 32
    iota = lax.iota(jnp.int32, 16)

    def per_query(ql, _):
        q = wid * nper + ql
        pltpu.sync_copy(fi_hbm.at[q], fi_v)
        pltpu.sync_copy(w_hbm.at[q], w_v)
        fia = fi_v[pl.ds(0, 16)]
        fib = fi_v[pl.ds(16, 16)]
        descs = []
        for m in range(TOPM):
            fm = fia[m] if m < 16 else fib[m - 16]
            descs.append(pltpu.async_copy(
                series_hbm.at[pl.ds(fm * SER, SER)],
                sv.at[pl.ds(m * SER, SER)], sem))
        for d in descs:
            d.wait()
        for kk in range(SER // 16):
            def mb(m, acc):
                mv = lax.broadcast(m, (16,))
                wv = plsc.load_gather(w_v, [mv])
                vv = plsc.load_gather(sv, [mv * SER + 16 * kk + iota])
                return acc + wv * vv
            ob[pl.ds(16 * kk, 16)] = lax.fori_loop(
                0, TOPM, mb, jnp.zeros((16,), jnp.float32))
        pltpu.sync_copy(ob, out_hbm.at[q])
        return 0

    lax.fori_loop(0, nper, per_query, 0)


def _series_call(series2, fi_sc, w_sc):
    mesh = plsc.VectorSubcoreMesh(core_axis_name="c", subcore_axis_name="s")
    f = pl.kernel(
        _series_body,
        out_type=jax.ShapeDtypeStruct((B, SER), jnp.float32),
        mesh=mesh,
        compiler_params=pltpu.CompilerParams(needs_layout_passes=False),
        scratch_types=[
            pltpu.VMEM((32,), jnp.int32),
            pltpu.VMEM((32,), jnp.float32),
            pltpu.VMEM((TOPM * SER,), jnp.float32),
            pltpu.VMEM((SER,), jnp.float32),
            pltpu.SemaphoreType.DMA,
        ],
    )
    return f(series2, fi_sc, w_sc)


EMB_ORDER = [
    ('emb_dataset', 'dataset_id', None),
    ('emb_sensor', 'sensor_type_id', None),
    ('emb_location', 'physical_location_id', None),
    ('emb_hour', 'hour', (0, 23)),
    ('emb_weekday', 'day_of_week', (0, 6)),
    ('emb_month', 'month', (1, 12)),
    ('emb_week', 'week_of_year', (1, 53)),
    ('emb_season', 'season_id', (0, 3)),
    ('emb_holiday', 'is_holiday', (0, 1)),
    ('emb_peak', 'peak_status_id', (0, 1)),
    ('emb_regime', 'regime_id', (0, 15)),
    ('emb_event', 'event_id', (0, 7)),
    ('emb_trend', 'trend_state_id', (0, 2)),
    ('emb_vol', 'volatility_state_id', (0, 2)),
    ('emb_shape', 'shape_state_id', (0, 2)),
    ('emb_rel', 'reliability_id', (0, 2)),
    ('emb_mend', 'month_end_flag', (0, 1)),
]


def kernel(dataset_id, sensor_type_id, physical_location_id, hour, day_of_week,
           month, week_of_year, season_id, is_holiday, peak_status_id,
           regime_id, event_id, trend_state_id, volatility_state_id,
           shape_state_id, reliability_id, month_end_flag,
           exogenous_vars, local_state_by_period, pool_keys, pool_series,
           emb_dataset, emb_sensor, emb_location, emb_hour, emb_weekday,
           emb_month, emb_week, emb_season, emb_holiday, emb_peak,
           emb_regime, emb_event, emb_trend, emb_vol, emb_shape, emb_rel,
           emb_mend, cat_w1, cat_b1, cat_w2, cat_b2, exo_w1, exo_b1,
           exo_w2, exo_b2, loc_w1, loc_b1, loc_w2, loc_b2, ln_g, ln_b,
           gate_w1, gate_b1, gate_w2, gate_b2):
    kw = dict(locals())
    parts = []
    for emb, idn, clip in EMB_ORDER:
        ids = kw[idn]
        if clip is not None:
            ids = jnp.clip(ids, clip[0], clip[1])
        parts.append(jnp.take(kw[emb], ids, axis=0))
    cat = jnp.concatenate(parts, axis=1)
    ls12 = local_state_by_period.reshape(B, 12)
    r1 = lambda a: a.reshape(1, -1)

    qn = _ctx_call(cat, exogenous_vars, ls12,
                   cat_w1, r1(cat_b1), cat_w2, r1(cat_b2),
                   exo_w1, r1(exo_b1), exo_w2, r1(exo_b2),
                   loc_w1, r1(loc_b1), loc_w2, r1(loc_b2),
                   r1(ln_g), r1(ln_b))

    keys_pad = jnp.pad(pool_keys, ((0, KP - K), (0, 128 - CTX)))
    sim, m = _sim_call(qn, keys_pad)
    sim2 = sim.reshape(B * NCH, CHW)

    cv, ci, ck = _topk_call(m, sim2, keys_pad)

    wq = gate_w1[:CTX]
    wk = gate_w1[CTX:2 * CTX]
    wcv = gate_w1[2 * CTX].reshape(1, GH)
    wrk = gate_w1[2 * CTX + 1].reshape(1, GH)
    w2t = gate_w2.reshape(1, GH)
    wf, fif = _gate_call(qn, ck.reshape(B * COARSE, 128), cv,
                         ci.astype(jnp.float32),
                         wq, wk, wcv, wrk, r1(gate_b1), w2t,
                         gate_b2.reshape(1, 1))

    fi_sc = fif[:, :32].astype(jnp.int32)
    w_sc = wf[:, :32]
    out = _series_call(pool_series.reshape(-1), fi_sc, w_sc)
    return out.reshape(B, PRED, CH)


# speedup vs baseline: 3.3913x; 3.3913x over previous
import functools
import jax
import jax.numpy as jnp
from jax import lax
from jax.experimental import pallas as pl
from jax.experimental.pallas import tpu as pltpu
from jax.experimental.pallas import tpu_sc as plsc

B = 1024
K = 100000
CTX = 64
COARSE = 80
TOPM = 20
PRED = 96
CH = 7
TEMP = 0.1
ALPHA = 0.7
GH = 128

KP = 100352
CHW = 128
NCH = KP // CHW
QB_A = 256
QB_B = 256
KB_B = 2048
QB_D = 128
SER = PRED * CH
NEG = -1e30


def _gelu(x):
    return 0.5 * x * (1.0 + lax.erf(x * 0.7071067811865476))


def _bdot(a, b):
    return jnp.dot(a.astype(jnp.bfloat16), b.astype(jnp.bfloat16),
                   preferred_element_type=jnp.float32)


def _b(x):
    return x.astype(jnp.bfloat16).astype(jnp.float32)


def _ctx_body(cat_ref, exo_ref, ls_ref,
              cw1_ref, cb1_ref, cw2_ref, cb2_ref,
              ew1_ref, eb1_ref, ew2_ref, eb2_ref,
              lw1_ref, lb1_ref, lw2_ref, lb2_ref,
              g_ref, b_ref, qn_ref):
    cat = cat_ref[...]
    cc = _bdot(_gelu(_bdot(cat, cw1_ref[...]) + cb1_ref[...]),
               cw2_ref[...]) + cb2_ref[...]
    exo = _b(exo_ref[...])
    ew1 = _b(ew1_ref[...])
    eh = eb1_ref[...] + jnp.zeros((QB_A, CTX), jnp.float32)
    for j in range(8):
        eh = eh + exo[:, j:j + 1] * ew1[j:j + 1, :]
    eo = _bdot(_gelu(eh), ew2_ref[...]) + eb2_ref[...]
    cc = cc + eo
    ls = _b(ls_ref[...])
    lw1 = _b(lw1_ref[...])
    acc = jnp.zeros((QB_A, CTX), jnp.float32)
    for p in range(3):
        lh = lb1_ref[...] + jnp.zeros((QB_A, CTX), jnp.float32)
        for j in range(4):
            lh = lh + ls[:, 4 * p + j:4 * p + j + 1] * lw1[j:j + 1, :]
        lo = _bdot(_gelu(lh), lw2_ref[...]) + lb2_ref[...]
        h = cc + lo
        mu = jnp.mean(h, axis=1, keepdims=True)
        var = jnp.mean((h - mu) ** 2, axis=1, keepdims=True)
        acc = acc + ((h - mu) / jnp.sqrt(var + 1e-5) * g_ref[...] + b_ref[...])
    q = acc / 3.0
    qn = q / (jnp.sqrt(jnp.sum(q * q, axis=1, keepdims=True)) + 1e-8)
    qn_ref[...] = qn


def _ctx_call(cat, exo, ls12, cw1, cb1, cw2, cb2, ew1, eb1, ew2, eb2,
              lw1, lb1, lw2, lb2, g, b):
    nq = B // QB_A
    full = lambda shp: pl.BlockSpec(shp, lambda i: (0, 0))
    return pl.pallas_call(
        _ctx_body,
        grid=(nq,),
        in_specs=[
            pl.BlockSpec((QB_A, 272), lambda i: (i, 0)),
            pl.BlockSpec((QB_A, 8), lambda i: (i, 0)),
            pl.BlockSpec((QB_A, 12), lambda i: (i, 0)),
            full((272, CTX)), full((1, CTX)), full((CTX, CTX)), full((1, CTX)),
            full((8, CTX)), full((1, CTX)), full((CTX, CTX)), full((1, CTX)),
            full((4, CTX)), full((1, CTX)), full((CTX, CTX)), full((1, CTX)),
            full((1, CTX)), full((1, CTX)),
        ],
        out_specs=pl.BlockSpec((QB_A, CTX), lambda i: (i, 0)),
        out_shape=jax.ShapeDtypeStruct((B, CTX), jnp.float32),
    )(cat, exo, ls12, cw1, cb1, cw2, cb2, ew1, eb1, ew2, eb2,
      lw1, lb1, lw2, lb2, g, b)


def _sim_body(qn_ref, keys_ref, sim_ref, m_ref):
    k = pl.program_id(0)
    keys = keys_ref[...][:, :CTX]
    nrm = jnp.sqrt(jnp.sum(keys * keys, axis=1, keepdims=True))
    kn = keys / (nrm + 1e-8)
    sim = lax.dot_general(qn_ref[...].astype(jnp.bfloat16),
                          kn.astype(jnp.bfloat16), (((1,), (1,)), ((), ())),
                          preferred_element_type=jnp.float32)
    col = k * KB_B + lax.broadcasted_iota(jnp.int32, (QB_B, KB_B), 1)
    sim = jnp.where(col < K, sim, NEG)
    sim3 = sim.reshape(QB_B, KB_B // CHW, CHW)
    sim_ref[...] = sim3
    m_ref[...] = jnp.max(sim3, axis=2).reshape(1, QB_B, KB_B // CHW)


def _sim_call(qn, keys_pad):
    nk = KP // KB_B
    nq = B // QB_B
    return pl.pallas_call(
        _sim_body,
        grid=(nk, nq),
        in_specs=[
            pl.BlockSpec((QB_B, CTX), lambda k, q: (q, 0)),
            pl.BlockSpec((KB_B, 128), lambda k, q: (k, 0)),
        ],
        out_specs=[
            pl.BlockSpec((QB_B, KB_B // CHW, CHW), lambda k, q: (q, k, 0)),
            pl.BlockSpec((1, QB_B, KB_B // CHW), lambda k, q: (k, q, 0)),
        ],
        out_shape=[
            jax.ShapeDtypeStruct((B, NCH, CHW), jnp.float32),
            jax.ShapeDtypeStruct((KP // KB_B, B, KB_B // CHW), jnp.float32),
        ],
    )(qn, keys_pad)


def _splat_f(x):
    return lax.broadcast(x, (16,))


def _topk_body(m_hbm, sim_hbm, keys_hbm, cv_hbm, ci_hbm, ck_hbm, *bufs):
    b0 = bufs[0:11]
    b1 = bufs[11:22]
    sem0, sem1 = bufs[22], bufs[23]
    wid = lax.axis_index("s") * 2 + lax.axis_index("c")
    nper = B // 32
    iota = lax.iota(jnp.int32, 16)
    lane0 = iota == 0
    negv = jnp.full((16,), NEG, jnp.float32)
    big = jnp.int32(1 << 20)

    for m_v, gm_v, cidx_v, absidx_v, cm_v, ch_v, cv_v, ci_v, ci80_v, ck_v, \
            _sub in (b0, b1):
        for i in range(8):
            cv_v[pl.ds(16 * i, 16)] = negv
            ci_v[pl.ds(16 * i, 16)] = jnp.zeros((16,), jnp.int32)
        cm_v[pl.ds(80, 16)] = negv
        gm_v[pl.ds(48, 16)] = negv

    def per_pair(pi, _):
        q0 = wid * nper + 2 * pi
        qs = (q0, q0 + 1)
        d0 = pltpu.async_copy(m_hbm.at[:, qs[0]], b0[0], sem0)
        d1 = pltpu.async_copy(m_hbm.at[:, qs[1]], b1[0], sem1)
        d0.wait()
        d1.wait()
        for j in range(NCH // 16):
            jv = lax.broadcast(jnp.int32(j), (16,))
            for (m_v, gm_v, *_r) in (b0, b1):
                plsc.store_scatter(gm_v, [jv], _splat_f(jnp.max(m_v[j, :])),
                                   mask=lane0)

        def sel_body(t, _c):
            tv = lax.broadcast(t, (16,))
            for (m_v, gm_v, cidx_v, _a, cm_v, *_r) in (b0, b1):
                g = jnp.full((16,), NEG, jnp.float32)
                for i in range(4):
                    g = jnp.maximum(g, gm_v[pl.ds(16 * i, 16)])
                s = jnp.max(g)
                jst = big
                for i in range(4):
                    v = gm_v[pl.ds(16 * i, 16)]
                    jst = jnp.minimum(jst, jnp.min(
                        jnp.where(v == s, iota + 16 * i, big)))
                jv = lax.broadcast(jst, (16,))
                mrow = plsc.load_gather(m_v, [jv, iota])
                lst = jnp.min(jnp.where(mrow == s, iota, big))
                cst = jst * 16 + lst
                plsc.store_scatter(cidx_v, [tv], lax.broadcast(cst, (16,)),
                                   mask=lane0)
                plsc.store_scatter(cm_v, [tv], _splat_f(s), mask=lane0)
                plsc.store_scatter(m_v, [jv, lax.broadcast(lst, (16,))],
                                   negv, mask=lane0)
                mrow2 = plsc.load_gather(m_v, [jv, iota])
                plsc.store_scatter(gm_v, [jv], _splat_f(jnp.max(mrow2)),
                                   mask=lane0)
            return 0

        lax.fori_loop(0, COARSE, sel_body, 0)

        for i in range(5):
            b0[3][pl.ds(16 * i, 16)] = b0[2][pl.ds(16 * i, 16)] + qs[0] * NCH
            b1[3][pl.ds(16 * i, 16)] = b1[2][pl.ds(16 * i, 16)] + qs[1] * NCH
        g0 = pltpu.async_copy(sim_hbm.at[b0[3]], b0[5], sem0)
        g1 = pltpu.async_copy(sim_hbm.at[b1[3]], b1[5], sem1)
        g0.wait()
        g1.wait()

        i7 = jnp.bitwise_and(iota, 7)
        m8 = iota < 8

        def sub_body(p, _c):
            pv = lax.broadcast(p, (16,))
            for (bb) in (b0, b1):
                ch_v, sub_v = bb[5], bb[10]
                for kk in range(8):
                    v = plsc.load_gather(ch_v, [pv, 16 * kk + iota])
                    plsc.store_scatter(
                        sub_v, [lax.broadcast(p * 8 + kk, (16,))],
                        _splat_f(jnp.max(v)), mask=lane0)
            return 0

        lax.fori_loop(0, COARSE, sub_body, 0)

        def ext_body(t, _c):
            tv = lax.broadcast(t, (16,))
            for (_m, _g, cidx_v, _a, cm_v, ch_v, cv_v, ci_v, ci80_v, _ck,
                 sub_v) in (b0, b1):
                s = jnp.float32(NEG)
                for i in range(6):
                    s = jnp.maximum(s, jnp.max(cm_v[pl.ds(16 * i, 16)]))
                pst = big
                for i in range(6):
                    v = cm_v[pl.ds(16 * i, 16)]
                    pst = jnp.minimum(pst, jnp.min(
                        jnp.where(v == s, iota + 16 * i, big)))
                pv = lax.broadcast(pst, (16,))
                sm = plsc.load_gather(sub_v, [pst * 8 + i7])
                kst = jnp.min(jnp.where((sm == s) & m8, iota, big))
                kv = lax.broadcast(kst, (16,))
                v = plsc.load_gather(ch_v, [pv, kst * 16 + iota])
                lst = jnp.min(jnp.where(v == s, kst * 16 + iota, big))
                cch = jnp.max(plsc.load_gather(cidx_v, [pv]))
                gi = cch * CHW + lst
                plsc.store_scatter(cv_v, [tv], _splat_f(s), mask=lane0)
                plsc.store_scatter(ci_v, [tv], lax.broadcast(gi, (16,)),
                                   mask=lane0)
                plsc.store_scatter(ci80_v, [tv], lax.broadcast(gi, (16,)),
                                   mask=lane0)
                plsc.store_scatter(ch_v, [pv, lax.broadcast(lst, (16,))],
                                   negv, mask=lane0)
                v2 = plsc.load_gather(ch_v, [pv, kst * 16 + iota])
                plsc.store_scatter(sub_v, [pv * 8 + kv], _splat_f(jnp.max(v2)),
                                   mask=lane0)
                sm2 = plsc.load_gather(sub_v, [pst * 8 + i7])
                cmn = jnp.max(jnp.where(m8, sm2, NEG))
                plsc.store_scatter(cm_v, [pv], _splat_f(cmn), mask=lane0)
            return 0

        lax.fori_loop(0, COARSE, ext_body, 0)

        k0 = pltpu.async_copy(keys_hbm.at[b0[8]], b0[9], sem0)
        k1 = pltpu.async_copy(keys_hbm.at[b1[8]], b1[9], sem1)
        k0.wait()
        k1.wait()
        for bb, q in ((b0, qs[0]), (b1, qs[1])):
            pltpu.sync_copy(bb[6], cv_hbm.at[q])
            pltpu.sync_copy(bb[7], ci_hbm.at[q])
            pltpu.sync_copy(bb[9], ck_hbm.at[q])
        return 0

    lax.fori_loop(0, nper // 2, per_pair, 0)


def _topk_call(m, sim2, keys_pad):
    mesh = plsc.VectorSubcoreMesh(core_axis_name="c", subcore_axis_name="s")
    f = pl.kernel(
        _topk_body,
        out_type=[
            jax.ShapeDtypeStruct((B, 128), jnp.float32),
            jax.ShapeDtypeStruct((B, 128), jnp.int32),
            jax.ShapeDtypeStruct((B, COARSE, 128), jnp.float32),
        ],
        mesh=mesh,
        compiler_params=pltpu.CompilerParams(needs_layout_passes=False),
        scratch_types=[
            pltpu.VMEM((NCH // 16, 16), jnp.float32),
            pltpu.VMEM((64,), jnp.float32),
            pltpu.VMEM((COARSE,), jnp.int32),
            pltpu.VMEM((COARSE,), jnp.int32),
            pltpu.VMEM((96,), jnp.float32),
            pltpu.VMEM((COARSE, CHW), jnp.float32),
            pltpu.VMEM((128,), jnp.float32),
            pltpu.VMEM((128,), jnp.int32),
            pltpu.VMEM((COARSE,), jnp.int32),
            pltpu.VMEM((COARSE, 128), jnp.float32),
            pltpu.VMEM((COARSE * 8,), jnp.float32),
        ] * 2 + [
            pltpu.SemaphoreType.DMA,
            pltpu.SemaphoreType.DMA,
        ],
    )
    return f(m, sim2, keys_pad)


def _gate_body(qn_ref, ck_ref, cv_ref, cif_ref,
               wq_ref, wk_ref, wcv_ref, wrk_ref, b1_ref, w2_ref, b2_ref,
               wf_ref, fif_ref):
    ckf = ck_ref[...][:, :CTX]
    nrm = jnp.sqrt(jnp.sum(ckf * ckf, axis=1, keepdims=True))
    ckn = ckf / (nrm + 1e-8)
    hk = _bdot(ckn, wk_ref[...])
    hq = _bdot(qn_ref[...], wq_ref[...])
    h3 = hk.reshape(QB_D, COARSE, GH) + hq[:, None, :]
    cv = cv_ref[...][:, :COARSE]
    h3 = h3 + _b(cv)[..., None] * _b(wcv_ref[...])[None, :, :]
    rank = lax.broadcasted_iota(jnp.int32, (QB_D, COARSE, 1), 1).astype(
        jnp.float32) / COARSE
    h3 = h3 + _b(rank) * _b(wrk_ref[...])[None, :, :]
    h3 = h3 + b1_ref[...][None, :, :]
    g3 = _gelu(h3)
    gl = jnp.sum(_b(g3) * _b(w2_ref[...])[None, :, :], axis=2) + b2_ref[0, 0]
    gate = jax.nn.sigmoid(gl)
    fused = ALPHA * cv + (1.0 - ALPHA) * gate
    cif = cif_ref[...][:, :COARSE]
    lane = lax.broadcasted_iota(jnp.int32, (QB_D, COARSE), 1)
    cur = fused
    fvals = []
    fidx = []
    for t in range(TOPM):
        a = jnp.argmax(cur, axis=1)
        oh = lane == a[:, None]
        fvals.append(jnp.max(cur, axis=1))
        fidx.append(jnp.sum(jnp.where(oh, cif, 0.0), axis=1))
        cur = jnp.where(oh, NEG, cur)
    fv = jnp.stack(fvals, axis=1)
    fi = jnp.stack(fidx, axis=1)
    e = jnp.exp((fv - fv[:, :1]) / TEMP)
    w = e / jnp.sum(e, axis=1, keepdims=True)
    wf_ref[...] = jnp.concatenate(
        [w, jnp.zeros((QB_D, 128 - TOPM), jnp.float32)], axis=1)
    fif_ref[...] = jnp.concatenate(
        [fi, jnp.broadcast_to(fi[:, :1], (QB_D, 128 - TOPM))], axis=1)


def _gate_call(qn, ckflat, cv, cif, wq, wk, wcv, wrk, b1, w2t, b2):
    nq = B // QB_D
    full = lambda shp: pl.BlockSpec(shp, lambda i: (0, 0))
    return pl.pallas_call(
        _gate_body,
        grid=(nq,),
        in_specs=[
            pl.BlockSpec((QB_D, CTX), lambda i: (i, 0)),
            pl.BlockSpec((QB_D * COARSE, 128), lambda i: (i, 0)),
            pl.BlockSpec((QB_D, 128), lambda i: (i, 0)),
            pl.BlockSpec((QB_D, 128), lambda i: (i, 0)),
            full((CTX, GH)), full((CTX, GH)), full((1, GH)), full((1, GH)),
            full((1, GH)), full((1, GH)), full((1, 1)),
        ],
        out_specs=[
            pl.BlockSpec((QB_D, 128), lambda i: (i, 0)),
            pl.BlockSpec((QB_D, 128), lambda i: (i, 0)),
        ],
        out_shape=[
            jax.ShapeDtypeStruct((B, 128), jnp.float32),
            jax.ShapeDtypeStruct((B, 128), jnp.float32),
        ],
    )(qn, ckflat, cv, cif, wq, wk, wcv, wrk, b1, w2t, b2)


def _series_body(series_hbm, fi_hbm, w_hbm, out_hbm,
                 fi_v, w_v, sv, ob, sem):
    wid = lax.axis_index("s") * 2 + lax.axis_index("c")
    nper = B // 32
    iota = lax.iota(jnp.int32, 16)

    def per_query(ql, _):
        q = wid * nper + ql
        pltpu.sync_copy(fi_hbm.at[q], fi_v)
        pltpu.sync_copy(w_hbm.at[q], w_v)
        fia = fi_v[pl.ds(0, 16)]
        fib = fi_v[pl.ds(16, 16)]
        descs = []
        for m in range(TOPM):
            fm = fia[m] if m < 16 else fib[m - 16]
            descs.append(pltpu.async_copy(series_hbm.at[fm], sv.at[m], sem))
        for d in descs:
            d.wait()
        for kk in range(SER // 16):
            def mb(m, acc):
                mv = lax.broadcast(m, (16,))
                wv = plsc.load_gather(w_v, [mv])
                vv = plsc.load_gather(sv, [mv, 16 * kk + iota])
                return acc + wv * vv
            ob[pl.ds(16 * kk, 16)] = lax.fori_loop(
                0, TOPM, mb, jnp.zeros((16,), jnp.float32))
        pltpu.sync_copy(ob, out_hbm.at[q])
        return 0

    lax.fori_loop(0, nper, per_query, 0)


def _series_call(series2, fi_sc, w_sc):
    mesh = plsc.VectorSubcoreMesh(core_axis_name="c", subcore_axis_name="s")
    f = pl.kernel(
        _series_body,
        out_type=jax.ShapeDtypeStruct((B, SER), jnp.float32),
        mesh=mesh,
        compiler_params=pltpu.CompilerParams(needs_layout_passes=False),
        scratch_types=[
            pltpu.VMEM((32,), jnp.int32),
            pltpu.VMEM((32,), jnp.float32),
            pltpu.VMEM((TOPM, SER), jnp.float32),
            pltpu.VMEM((SER,), jnp.float32),
            pltpu.SemaphoreType.DMA,
        ],
    )
    return f(series2, fi_sc, w_sc)


EMB_ORDER = [
    ('emb_dataset', 'dataset_id', None),
    ('emb_sensor', 'sensor_type_id', None),
    ('emb_location', 'physical_location_id', None),
    ('emb_hour', 'hour', (0, 23)),
    ('emb_weekday', 'day_of_week', (0, 6)),
    ('emb_month', 'month', (1, 12)),
    ('emb_week', 'week_of_year', (1, 53)),
    ('emb_season', 'season_id', (0, 3)),
    ('emb_holiday', 'is_holiday', (0, 1)),
    ('emb_peak', 'peak_status_id', (0, 1)),
    ('emb_regime', 'regime_id', (0, 15)),
    ('emb_event', 'event_id', (0, 7)),
    ('emb_trend', 'trend_state_id', (0, 2)),
    ('emb_vol', 'volatility_state_id', (0, 2)),
    ('emb_shape', 'shape_state_id', (0, 2)),
    ('emb_rel', 'reliability_id', (0, 2)),
    ('emb_mend', 'month_end_flag', (0, 1)),
]


def kernel(dataset_id, sensor_type_id, physical_location_id, hour, day_of_week,
           month, week_of_year, season_id, is_holiday, peak_status_id,
           regime_id, event_id, trend_state_id, volatility_state_id,
           shape_state_id, reliability_id, month_end_flag,
           exogenous_vars, local_state_by_period, pool_keys, pool_series,
           emb_dataset, emb_sensor, emb_location, emb_hour, emb_weekday,
           emb_month, emb_week, emb_season, emb_holiday, emb_peak,
           emb_regime, emb_event, emb_trend, emb_vol, emb_shape, emb_rel,
           emb_mend, cat_w1, cat_b1, cat_w2, cat_b2, exo_w1, exo_b1,
           exo_w2, exo_b2, loc_w1, loc_b1, loc_w2, loc_b2, ln_g, ln_b,
           gate_w1, gate_b1, gate_w2, gate_b2):
    kw = dict(locals())
    parts = []
    for emb, idn, clip in EMB_ORDER:
        ids = kw[idn]
        if clip is not None:
            ids = jnp.clip(ids, clip[0], clip[1])
        parts.append(jnp.take(kw[emb], ids, axis=0))
    cat = jnp.concatenate(parts, axis=1)
    ls12 = local_state_by_period.reshape(B, 12)
    r1 = lambda a: a.reshape(1, -1)

    qn = _ctx_call(cat, exogenous_vars, ls12,
                   cat_w1, r1(cat_b1), cat_w2, r1(cat_b2),
                   exo_w1, r1(exo_b1), exo_w2, r1(exo_b2),
                   loc_w1, r1(loc_b1), loc_w2, r1(loc_b2),
                   r1(ln_g), r1(ln_b))

    keys_pad = jnp.pad(pool_keys, ((0, KP - K), (0, 128 - CTX)))
    sim, m = _sim_call(qn, keys_pad)
    sim2 = sim.reshape(B * NCH, CHW)

    cv, ci, ck = _topk_call(m, sim2, keys_pad)

    wq = gate_w1[:CTX]
    wk = gate_w1[CTX:2 * CTX]
    wcv = gate_w1[2 * CTX].reshape(1, GH)
    wrk = gate_w1[2 * CTX + 1].reshape(1, GH)
    w2t = gate_w2.reshape(1, GH)
    wf, fif = _gate_call(qn, ck.reshape(B * COARSE, 128), cv,
                         ci.astype(jnp.float32),
                         wq, wk, wcv, wrk, r1(gate_b1), w2t,
                         gate_b2.reshape(1, 1))

    fi_sc = fif[:, :32].astype(jnp.int32)
    w_sc = wf[:, :32]
    out = _series_call(pool_series.reshape(K, SER), fi_sc, w_sc)
    return out.reshape(B, PRED, CH)


# speedup vs baseline: 3.9397x; 1.1617x over previous
import functools
import jax
import jax.numpy as jnp
from jax import lax
from jax.experimental import pallas as pl
from jax.experimental.pallas import tpu as pltpu
from jax.experimental.pallas import tpu_sc as plsc

B = 1024
K = 100000
CTX = 64
COARSE = 80
TOPM = 20
PRED = 96
CH = 7
TEMP = 0.1
ALPHA = 0.7
GH = 128

KP = 100352
CHW = 128
NCH = KP // CHW
QB_A = 256
QB_B = 256
KB_B = 2048
QB_D = 128
SER = PRED * CH
NEG = -1e30


def _gelu(x):
    return 0.5 * x * (1.0 + lax.erf(x * 0.7071067811865476))


def _bdot(a, b):
    return jnp.dot(a.astype(jnp.bfloat16), b.astype(jnp.bfloat16),
                   preferred_element_type=jnp.float32)


def _b(x):
    return x.astype(jnp.bfloat16).astype(jnp.float32)


def _ctx_body(cat_ref, exo_ref, ls_ref,
              cw1_ref, cb1_ref, cw2_ref, cb2_ref,
              ew1_ref, eb1_ref, ew2_ref, eb2_ref,
              lw1_ref, lb1_ref, lw2_ref, lb2_ref,
              g_ref, b_ref, qn_ref):
    cat = cat_ref[...]
    cc = _bdot(_gelu(_bdot(cat, cw1_ref[...]) + cb1_ref[...]),
               cw2_ref[...]) + cb2_ref[...]
    exo = _b(exo_ref[...])
    ew1 = _b(ew1_ref[...])
    eh = eb1_ref[...] + jnp.zeros((QB_A, CTX), jnp.float32)
    for j in range(8):
        eh = eh + exo[:, j:j + 1] * ew1[j:j + 1, :]
    eo = _bdot(_gelu(eh), ew2_ref[...]) + eb2_ref[...]
    cc = cc + eo
    ls = _b(ls_ref[...])
    lw1 = _b(lw1_ref[...])
    acc = jnp.zeros((QB_A, CTX), jnp.float32)
    for p in range(3):
        lh = lb1_ref[...] + jnp.zeros((QB_A, CTX), jnp.float32)
        for j in range(4):
            lh = lh + ls[:, 4 * p + j:4 * p + j + 1] * lw1[j:j + 1, :]
        lo = _bdot(_gelu(lh), lw2_ref[...]) + lb2_ref[...]
        h = cc + lo
        mu = jnp.mean(h, axis=1, keepdims=True)
        var = jnp.mean((h - mu) ** 2, axis=1, keepdims=True)
        acc = acc + ((h - mu) / jnp.sqrt(var + 1e-5) * g_ref[...] + b_ref[...])
    q = acc / 3.0
    qn = q / (jnp.sqrt(jnp.sum(q * q, axis=1, keepdims=True)) + 1e-8)
    qn_ref[...] = qn


def _ctx_call(cat, exo, ls12, cw1, cb1, cw2, cb2, ew1, eb1, ew2, eb2,
              lw1, lb1, lw2, lb2, g, b):
    nq = B // QB_A
    full = lambda shp: pl.BlockSpec(shp, lambda i: (0, 0))
    return pl.pallas_call(
        _ctx_body,
        grid=(nq,),
        in_specs=[
            pl.BlockSpec((QB_A, 272), lambda i: (i, 0)),
            pl.BlockSpec((QB_A, 8), lambda i: (i, 0)),
            pl.BlockSpec((QB_A, 12), lambda i: (i, 0)),
            full((272, CTX)), full((1, CTX)), full((CTX, CTX)), full((1, CTX)),
            full((8, CTX)), full((1, CTX)), full((CTX, CTX)), full((1, CTX)),
            full((4, CTX)), full((1, CTX)), full((CTX, CTX)), full((1, CTX)),
            full((1, CTX)), full((1, CTX)),
        ],
        out_specs=pl.BlockSpec((QB_A, CTX), lambda i: (i, 0)),
        out_shape=jax.ShapeDtypeStruct((B, CTX), jnp.float32),
    )(cat, exo, ls12, cw1, cb1, cw2, cb2, ew1, eb1, ew2, eb2,
      lw1, lb1, lw2, lb2, g, b)


def _sim_body(qn_ref, keys_ref, sim_ref, m_ref):
    k = pl.program_id(0)
    keys = keys_ref[...][:, :CTX]
    nrm = jnp.sqrt(jnp.sum(keys * keys, axis=1, keepdims=True))
    kn = keys / (nrm + 1e-8)
    sim = lax.dot_general(qn_ref[...].astype(jnp.bfloat16),
                          kn.astype(jnp.bfloat16), (((1,), (1,)), ((), ())),
                          preferred_element_type=jnp.float32)
    col = k * KB_B + lax.broadcasted_iota(jnp.int32, (QB_B, KB_B), 1)
    sim = jnp.where(col < K, sim, NEG)
    sim3 = sim.reshape(QB_B, KB_B // CHW, CHW)
    sim_ref[...] = sim3
    m_ref[...] = jnp.max(sim3, axis=2).reshape(1, QB_B, KB_B // CHW)


def _sim_call(qn, keys_pad):
    nk = KP // KB_B
    nq = B // QB_B
    return pl.pallas_call(
        _sim_body,
        grid=(nk, nq),
        in_specs=[
            pl.BlockSpec((QB_B, CTX), lambda k, q: (q, 0)),
            pl.BlockSpec((KB_B, 128), lambda k, q: (k, 0)),
        ],
        out_specs=[
            pl.BlockSpec((QB_B, KB_B // CHW, CHW), lambda k, q: (q, k, 0)),
            pl.BlockSpec((1, QB_B, KB_B // CHW), lambda k, q: (k, q, 0)),
        ],
        out_shape=[
            jax.ShapeDtypeStruct((B, NCH, CHW), jnp.float32),
            jax.ShapeDtypeStruct((KP // KB_B, B, KB_B // CHW), jnp.float32),
        ],
    )(qn, keys_pad)


def _splat_f(x):
    return lax.broadcast(x, (16,))


def _topk_body(m_hbm, sim_hbm, keys_hbm, cv_hbm, ci_hbm, ck_hbm, *bufs):
    b0 = bufs[0:10]
    b1 = bufs[10:20]
    sem0, sem1 = bufs[20], bufs[21]
    wid = lax.axis_index("s") * 2 + lax.axis_index("c")
    nper = B // 32
    iota = lax.iota(jnp.int32, 16)
    lane0 = iota == 0
    negv = jnp.full((16,), NEG, jnp.float32)
    big = jnp.int32(1 << 20)

    for m_v, gm_v, cidx_v, absidx_v, cm_v, ch_v, cv_v, ci_v, ci80_v, ck_v \
            in (b0, b1):
        for i in range(8):
            cv_v[pl.ds(16 * i, 16)] = negv
            ci_v[pl.ds(16 * i, 16)] = jnp.zeros((16,), jnp.int32)
        cm_v[pl.ds(80, 16)] = negv
        gm_v[pl.ds(48, 16)] = negv

    def per_pair(pi, _):
        q0 = wid * nper + 2 * pi
        qs = (q0, q0 + 1)
        d0 = pltpu.async_copy(m_hbm.at[:, qs[0]], b0[0], sem0)
        d1 = pltpu.async_copy(m_hbm.at[:, qs[1]], b1[0], sem1)
        d0.wait()
        d1.wait()
        for j in range(NCH // 16):
            jv = lax.broadcast(jnp.int32(j), (16,))
            for (m_v, gm_v, *_r) in (b0, b1):
                plsc.store_scatter(gm_v, [jv], _splat_f(jnp.max(m_v[j, :])),
                                   mask=lane0)

        def sel_body(t, _c):
            tv = lax.broadcast(t, (16,))
            for (m_v, gm_v, cidx_v, _a, cm_v, *_r) in (b0, b1):
                g = jnp.full((16,), NEG, jnp.float32)
                for i in range(4):
                    g = jnp.maximum(g, gm_v[pl.ds(16 * i, 16)])
                s = jnp.max(g)
                jst = big
                for i in range(4):
                    v = gm_v[pl.ds(16 * i, 16)]
                    jst = jnp.minimum(jst, jnp.min(
                        jnp.where(v == s, iota + 16 * i, big)))
                jv = lax.broadcast(jst, (16,))
                mrow = plsc.load_gather(m_v, [jv, iota])
                lst = jnp.min(jnp.where(mrow == s, iota, big))
                cst = jst * 16 + lst
                plsc.store_scatter(cidx_v, [tv], lax.broadcast(cst, (16,)),
                                   mask=lane0)
                plsc.store_scatter(cm_v, [tv], _splat_f(s), mask=lane0)
                plsc.store_scatter(m_v, [jv, lax.broadcast(lst, (16,))],
                                   negv, mask=lane0)
                mrow2 = plsc.load_gather(m_v, [jv, iota])
                plsc.store_scatter(gm_v, [jv], _splat_f(jnp.max(mrow2)),
                                   mask=lane0)
            return 0

        lax.fori_loop(0, COARSE, sel_body, 0)

        for i in range(5):
            b0[3][pl.ds(16 * i, 16)] = b0[2][pl.ds(16 * i, 16)] + qs[0] * NCH
            b1[3][pl.ds(16 * i, 16)] = b1[2][pl.ds(16 * i, 16)] + qs[1] * NCH
        g0 = pltpu.async_copy(sim_hbm.at[b0[3]], b0[5], sem0)
        g1 = pltpu.async_copy(sim_hbm.at[b1[3]], b1[5], sem1)
        g0.wait()
        g1.wait()

        def ext_body(t, _c):
            tv = lax.broadcast(t, (16,))
            for (_m, _g, cidx_v, _a, cm_v, ch_v, cv_v, ci_v, ci80_v, _ck) \
                    in (b0, b1):
                s = jnp.float32(NEG)
                for i in range(6):
                    s = jnp.maximum(s, jnp.max(cm_v[pl.ds(16 * i, 16)]))
                pst = big
                for i in range(6):
                    v = cm_v[pl.ds(16 * i, 16)]
                    pst = jnp.minimum(pst, jnp.min(
                        jnp.where(v == s, iota + 16 * i, big)))
                pv = lax.broadcast(pst, (16,))
                lst = big
                for kk in range(8):
                    v = plsc.load_gather(ch_v, [pv, 16 * kk + iota])
                    lst = jnp.minimum(lst, jnp.min(
                        jnp.where(v == s, 16 * kk + iota, big)))
                cch = jnp.max(plsc.load_gather(cidx_v, [pv]))
                gi = cch * CHW + lst
                plsc.store_scatter(cv_v, [tv], _splat_f(s), mask=lane0)
                plsc.store_scatter(ci_v, [tv], lax.broadcast(gi, (16,)),
                                   mask=lane0)
                plsc.store_scatter(ci80_v, [tv], lax.broadcast(gi, (16,)),
                                   mask=lane0)
                plsc.store_scatter(ch_v, [pv, lax.broadcast(lst, (16,))],
                                   negv, mask=lane0)
                nm = jnp.full((16,), NEG, jnp.float32)
                for kk in range(8):
                    nm = jnp.maximum(nm, plsc.load_gather(
                        ch_v, [pv, 16 * kk + iota]))
                plsc.store_scatter(cm_v, [pv], _splat_f(jnp.max(nm)),
                                   mask=lane0)
            return 0

        lax.fori_loop(0, COARSE, ext_body, 0)

        k0 = pltpu.async_copy(keys_hbm.at[b0[8]], b0[9], sem0)
        k1 = pltpu.async_copy(keys_hbm.at[b1[8]], b1[9], sem1)
        k0.wait()
        k1.wait()
        for bb, q in ((b0, qs[0]), (b1, qs[1])):
            pltpu.sync_copy(bb[6], cv_hbm.at[q])
            pltpu.sync_copy(bb[7], ci_hbm.at[q])
            pltpu.sync_copy(bb[9], ck_hbm.at[q])
        return 0

    lax.fori_loop(0, nper // 2, per_pair, 0)


def _topk_call(m, sim2, keys_pad):
    mesh = plsc.VectorSubcoreMesh(core_axis_name="c", subcore_axis_name="s")
    f = pl.kernel(
        _topk_body,
        out_type=[
            jax.ShapeDtypeStruct((B, 128), jnp.float32),
            jax.ShapeDtypeStruct((B, 128), jnp.int32),
            jax.ShapeDtypeStruct((B, COARSE, 128), jnp.float32),
        ],
        mesh=mesh,
        compiler_params=pltpu.CompilerParams(needs_layout_passes=False),
        scratch_types=[
            pltpu.VMEM((NCH // 16, 16), jnp.float32),
            pltpu.VMEM((64,), jnp.float32),
            pltpu.VMEM((COARSE,), jnp.int32),
            pltpu.VMEM((COARSE,), jnp.int32),
            pltpu.VMEM((96,), jnp.float32),
            pltpu.VMEM((COARSE, CHW), jnp.float32),
            pltpu.VMEM((128,), jnp.float32),
            pltpu.VMEM((128,), jnp.int32),
            pltpu.VMEM((COARSE,), jnp.int32),
            pltpu.VMEM((COARSE, 128), jnp.float32),
        ] * 2 + [
            pltpu.SemaphoreType.DMA,
            pltpu.SemaphoreType.DMA,
        ],
    )
    return f(m, sim2, keys_pad)


def _gate_body(qn_ref, ck_ref, cv_ref, cif_ref,
               wq_ref, wk_ref, wcv_ref, wrk_ref, b1_ref, w2_ref, b2_ref,
               wf_ref, fif_ref):
    ckf = ck_ref[...][:, :CTX]
    nrm = jnp.sqrt(jnp.sum(ckf * ckf, axis=1, keepdims=True))
    ckn = ckf / (nrm + 1e-8)
    hk = _bdot(ckn, wk_ref[...])
    hq = _bdot(qn_ref[...], wq_ref[...])
    h3 = hk.reshape(QB_D, COARSE, GH) + hq[:, None, :]
    cv = cv_ref[...][:, :COARSE]
    h3 = h3 + _b(cv)[..., None] * _b(wcv_ref[...])[None, :, :]
    rank = lax.broadcasted_iota(jnp.int32, (QB_D, COARSE, 1), 1).astype(
        jnp.float32) / COARSE
    h3 = h3 + _b(rank) * _b(wrk_ref[...])[None, :, :]
    h3 = h3 + b1_ref[...][None, :, :]
    g3 = _gelu(h3)
    gl = jnp.sum(_b(g3) * _b(w2_ref[...])[None, :, :], axis=2) + b2_ref[0, 0]
    gate = jax.nn.sigmoid(gl)
    fused = ALPHA * cv + (1.0 - ALPHA) * gate
    cif = cif_ref[...][:, :COARSE]
    lane = lax.broadcasted_iota(jnp.int32, (QB_D, COARSE), 1)
    cur = fused
    fvals = []
    fidx = []
    for t in range(TOPM):
        a = jnp.argmax(cur, axis=1)
        oh = lane == a[:, None]
        fvals.append(jnp.max(cur, axis=1))
        fidx.append(jnp.sum(jnp.where(oh, cif, 0.0), axis=1))
        cur = jnp.where(oh, NEG, cur)
    fv = jnp.stack(fvals, axis=1)
    fi = jnp.stack(fidx, axis=1)
    e = jnp.exp((fv - fv[:, :1]) / TEMP)
    w = e / jnp.sum(e, axis=1, keepdims=True)
    wf_ref[...] = jnp.concatenate(
        [w, jnp.zeros((QB_D, 128 - TOPM), jnp.float32)], axis=1)
    fif_ref[...] = jnp.concatenate(
        [fi, jnp.broadcast_to(fi[:, :1], (QB_D, 128 - TOPM))], axis=1)


def _gate_call(qn, ckflat, cv, cif, wq, wk, wcv, wrk, b1, w2t, b2):
    nq = B // QB_D
    full = lambda shp: pl.BlockSpec(shp, lambda i: (0, 0))
    return pl.pallas_call(
        _gate_body,
        grid=(nq,),
        in_specs=[
            pl.BlockSpec((QB_D, CTX), lambda i: (i, 0)),
            pl.BlockSpec((QB_D * COARSE, 128), lambda i: (i, 0)),
            pl.BlockSpec((QB_D, 128), lambda i: (i, 0)),
            pl.BlockSpec((QB_D, 128), lambda i: (i, 0)),
            full((CTX, GH)), full((CTX, GH)), full((1, GH)), full((1, GH)),
            full((1, GH)), full((1, GH)), full((1, 1)),
        ],
        out_specs=[
            pl.BlockSpec((QB_D, 128), lambda i: (i, 0)),
            pl.BlockSpec((QB_D, 128), lambda i: (i, 0)),
        ],
        out_shape=[
            jax.ShapeDtypeStruct((B, 128), jnp.float32),
            jax.ShapeDtypeStruct((B, 128), jnp.float32),
        ],
    )(qn, ckflat, cv, cif, wq, wk, wcv, wrk, b1, w2t, b2)


def _series_body(series_hbm, fi_hbm, w_hbm, out_hbm,
                 fi_v, w_v, sv, ob, sem):
    wid = lax.axis_index("s") * 2 + lax.axis_index("c")
    nper = B // 32
    iota = lax.iota(jnp.int32, 16)

    def per_query(ql, _):
        q = wid * nper + ql
        pltpu.sync_copy(fi_hbm.at[q], fi_v)
        pltpu.sync_copy(w_hbm.at[q], w_v)
        fia = fi_v[pl.ds(0, 16)]
        fib = fi_v[pl.ds(16, 16)]
        descs = []
        for m in range(TOPM):
            fm = fia[m] if m < 16 else fib[m - 16]
            descs.append(pltpu.async_copy(series_hbm.at[fm], sv.at[m], sem))
        for d in descs:
            d.wait()
        for kk in range(SER // 16):
            def mb(m, acc):
                mv = lax.broadcast(m, (16,))
                wv = plsc.load_gather(w_v, [mv])
                vv = plsc.load_gather(sv, [mv, 16 * kk + iota])
                return acc + wv * vv
            ob[pl.ds(16 * kk, 16)] = lax.fori_loop(
                0, TOPM, mb, jnp.zeros((16,), jnp.float32))
        pltpu.sync_copy(ob, out_hbm.at[q])
        return 0

    lax.fori_loop(0, nper, per_query, 0)


def _series_call(series2, fi_sc, w_sc):
    mesh = plsc.VectorSubcoreMesh(core_axis_name="c", subcore_axis_name="s")
    f = pl.kernel(
        _series_body,
        out_type=jax.ShapeDtypeStruct((B, SER), jnp.float32),
        mesh=mesh,
        compiler_params=pltpu.CompilerParams(needs_layout_passes=False),
        scratch_types=[
            pltpu.VMEM((32,), jnp.int32),
            pltpu.VMEM((32,), jnp.float32),
            pltpu.VMEM((TOPM, SER), jnp.float32),
            pltpu.VMEM((SER,), jnp.float32),
            pltpu.SemaphoreType.DMA,
        ],
    )
    return f(series2, fi_sc, w_sc)


EMB_ORDER = [
    ('emb_dataset', 'dataset_id', None),
    ('emb_sensor', 'sensor_type_id', None),
    ('emb_location', 'physical_location_id', None),
    ('emb_hour', 'hour', (0, 23)),
    ('emb_weekday', 'day_of_week', (0, 6)),
    ('emb_month', 'month', (1, 12)),
    ('emb_week', 'week_of_year', (1, 53)),
    ('emb_season', 'season_id', (0, 3)),
    ('emb_holiday', 'is_holiday', (0, 1)),
    ('emb_peak', 'peak_status_id', (0, 1)),
    ('emb_regime', 'regime_id', (0, 15)),
    ('emb_event', 'event_id', (0, 7)),
    ('emb_trend', 'trend_state_id', (0, 2)),
    ('emb_vol', 'volatility_state_id', (0, 2)),
    ('emb_shape', 'shape_state_id', (0, 2)),
    ('emb_rel', 'reliability_id', (0, 2)),
    ('emb_mend', 'month_end_flag', (0, 1)),
]


def kernel(dataset_id, sensor_type_id, physical_location_id, hour, day_of_week,
           month, week_of_year, season_id, is_holiday, peak_status_id,
           regime_id, event_id, trend_state_id, volatility_state_id,
           shape_state_id, reliability_id, month_end_flag,
           exogenous_vars, local_state_by_period, pool_keys, pool_series,
           emb_dataset, emb_sensor, emb_location, emb_hour, emb_weekday,
           emb_month, emb_week, emb_season, emb_holiday, emb_peak,
           emb_regime, emb_event, emb_trend, emb_vol, emb_shape, emb_rel,
           emb_mend, cat_w1, cat_b1, cat_w2, cat_b2, exo_w1, exo_b1,
           exo_w2, exo_b2, loc_w1, loc_b1, loc_w2, loc_b2, ln_g, ln_b,
           gate_w1, gate_b1, gate_w2, gate_b2):
    kw = dict(locals())
    parts = []
    for emb, idn, clip in EMB_ORDER:
        ids = kw[idn]
        if clip is not None:
            ids = jnp.clip(ids, clip[0], clip[1])
        parts.append(jnp.take(kw[emb], ids, axis=0))
    cat = jnp.concatenate(parts, axis=1)
    ls12 = local_state_by_period.reshape(B, 12)
    r1 = lambda a: a.reshape(1, -1)

    qn = _ctx_call(cat, exogenous_vars, ls12,
                   cat_w1, r1(cat_b1), cat_w2, r1(cat_b2),
                   exo_w1, r1(exo_b1), exo_w2, r1(exo_b2),
                   loc_w1, r1(loc_b1), loc_w2, r1(loc_b2),
                   r1(ln_g), r1(ln_b))

    keys_pad = jnp.pad(pool_keys, ((0, KP - K), (0, 128 - CTX)))
    sim, m = _sim_call(qn, keys_pad)
    sim2 = sim.reshape(B * NCH, CHW)

    cv, ci, ck = _topk_call(m, sim2, keys_pad)

    wq = gate_w1[:CTX]
    wk = gate_w1[CTX:2 * CTX]
    wcv = gate_w1[2 * CTX].reshape(1, GH)
    wrk = gate_w1[2 * CTX + 1].reshape(1, GH)
    w2t = gate_w2.reshape(1, GH)
    wf, fif = _gate_call(qn, ck.reshape(B * COARSE, 128), cv,
                         ci.astype(jnp.float32),
                         wq, wk, wcv, wrk, r1(gate_b1), w2t,
                         gate_b2.reshape(1, 1))

    fi_sc = fif[:, :32].astype(jnp.int32)
    w_sc = wf[:, :32]
    out = _series_call(pool_series.reshape(K, SER), fi_sc, w_sc)
    return out.reshape(B, PRED, CH)


# speedup vs baseline: 4.1749x; 1.0597x over previous
import functools
import jax
import jax.numpy as jnp
from jax import lax
from jax.experimental import pallas as pl
from jax.experimental.pallas import tpu as pltpu
from jax.experimental.pallas import tpu_sc as plsc

B = 1024
K = 100000
CTX = 64
COARSE = 80
TOPM = 20
PRED = 96
CH = 7
TEMP = 0.1
ALPHA = 0.7
GH = 128

KP = 100352
CHW = 128
NCH = KP // CHW
QB_A = 256
QB_B = 256
KB_B = 2048
QB_D = 128
SER = PRED * CH
NEG = -1e30


def _gelu(x):
    return 0.5 * x * (1.0 + lax.erf(x * 0.7071067811865476))


def _bdot(a, b):
    return jnp.dot(a.astype(jnp.bfloat16), b.astype(jnp.bfloat16),
                   preferred_element_type=jnp.float32)


def _b(x):
    return x.astype(jnp.bfloat16).astype(jnp.float32)


def _ctx_body(cat_ref, exo_ref, ls_ref,
              cw1_ref, cb1_ref, cw2_ref, cb2_ref,
              ew1_ref, eb1_ref, ew2_ref, eb2_ref,
              lw1_ref, lb1_ref, lw2_ref, lb2_ref,
              g_ref, b_ref, qn_ref):
    cat = cat_ref[...]
    cc = _bdot(_gelu(_bdot(cat, cw1_ref[...]) + cb1_ref[...]),
               cw2_ref[...]) + cb2_ref[...]
    exo = _b(exo_ref[...])
    ew1 = _b(ew1_ref[...])
    eh = eb1_ref[...] + jnp.zeros((QB_A, CTX), jnp.float32)
    for j in range(8):
        eh = eh + exo[:, j:j + 1] * ew1[j:j + 1, :]
    eo = _bdot(_gelu(eh), ew2_ref[...]) + eb2_ref[...]
    cc = cc + eo
    ls = _b(ls_ref[...])
    lw1 = _b(lw1_ref[...])
    acc = jnp.zeros((QB_A, CTX), jnp.float32)
    for p in range(3):
        lh = lb1_ref[...] + jnp.zeros((QB_A, CTX), jnp.float32)
        for j in range(4):
            lh = lh + ls[:, 4 * p + j:4 * p + j + 1] * lw1[j:j + 1, :]
        lo = _bdot(_gelu(lh), lw2_ref[...]) + lb2_ref[...]
        h = cc + lo
        mu = jnp.mean(h, axis=1, keepdims=True)
        var = jnp.mean((h - mu) ** 2, axis=1, keepdims=True)
        acc = acc + ((h - mu) / jnp.sqrt(var + 1e-5) * g_ref[...] + b_ref[...])
    q = acc / 3.0
    qn = q / (jnp.sqrt(jnp.sum(q * q, axis=1, keepdims=True)) + 1e-8)
    qn_ref[...] = qn


def _ctx_call(cat, exo, ls12, cw1, cb1, cw2, cb2, ew1, eb1, ew2, eb2,
              lw1, lb1, lw2, lb2, g, b):
    nq = B // QB_A
    full = lambda shp: pl.BlockSpec(shp, lambda i: (0, 0))
    return pl.pallas_call(
        _ctx_body,
        grid=(nq,),
        in_specs=[
            pl.BlockSpec((QB_A, 272), lambda i: (i, 0)),
            pl.BlockSpec((QB_A, 8), lambda i: (i, 0)),
            pl.BlockSpec((QB_A, 12), lambda i: (i, 0)),
            full((272, CTX)), full((1, CTX)), full((CTX, CTX)), full((1, CTX)),
            full((8, CTX)), full((1, CTX)), full((CTX, CTX)), full((1, CTX)),
            full((4, CTX)), full((1, CTX)), full((CTX, CTX)), full((1, CTX)),
            full((1, CTX)), full((1, CTX)),
        ],
        out_specs=pl.BlockSpec((QB_A, CTX), lambda i: (i, 0)),
        out_shape=jax.ShapeDtypeStruct((B, CTX), jnp.float32),
    )(cat, exo, ls12, cw1, cb1, cw2, cb2, ew1, eb1, ew2, eb2,
      lw1, lb1, lw2, lb2, g, b)


def _sim_body(qn_ref, keys_ref, sim_ref, m_ref):
    k = pl.program_id(0)
    keys = keys_ref[...][:, :CTX]
    nrm = jnp.sqrt(jnp.sum(keys * keys, axis=1, keepdims=True))
    kn = keys / (nrm + 1e-8)
    sim = lax.dot_general(qn_ref[...].astype(jnp.bfloat16),
                          kn.astype(jnp.bfloat16), (((1,), (1,)), ((), ())),
                          preferred_element_type=jnp.float32)
    col = k * KB_B + lax.broadcasted_iota(jnp.int32, (QB_B, KB_B), 1)
    sim = jnp.where(col < K, sim, NEG)
    sim3 = sim.reshape(QB_B, KB_B // CHW, CHW)
    sim_ref[...] = sim3
    m_ref[...] = jnp.max(sim3, axis=2).reshape(1, QB_B, KB_B // CHW)


def _sim_call(qn, keys_pad):
    nk = KP // KB_B
    nq = B // QB_B
    return pl.pallas_call(
        _sim_body,
        grid=(nk, nq),
        in_specs=[
            pl.BlockSpec((QB_B, CTX), lambda k, q: (q, 0)),
            pl.BlockSpec((KB_B, 128), lambda k, q: (k, 0)),
        ],
        out_specs=[
            pl.BlockSpec((QB_B, KB_B // CHW, CHW), lambda k, q: (q, k, 0)),
            pl.BlockSpec((1, QB_B, KB_B // CHW), lambda k, q: (k, q, 0)),
        ],
        out_shape=[
            jax.ShapeDtypeStruct((B, NCH, CHW), jnp.float32),
            jax.ShapeDtypeStruct((KP // KB_B, B, KB_B // CHW), jnp.float32),
        ],
    )(qn, keys_pad)


def _splat_f(x):
    return lax.broadcast(x, (16,))


def _topk_body(m_hbm, sim_hbm, keys_hbm, cv_hbm, ci_hbm, ck_hbm, *bufs):
    b0 = bufs[0:10]
    b1 = bufs[10:20]
    sem0, sem1 = bufs[20], bufs[21]
    wid = lax.axis_index("s") * 2 + lax.axis_index("c")
    nper = B // 32
    iota = lax.iota(jnp.int32, 16)
    lane0 = iota == 0
    negv = jnp.full((16,), NEG, jnp.float32)
    big = jnp.int32(1 << 20)

    for m_v, gm_v, cidx_v, absidx_v, cm_v, ch_v, cv_v, ci_v, ci80_v, ck_v \
            in (b0, b1):
        for i in range(8):
            cv_v[pl.ds(16 * i, 16)] = negv
            ci_v[pl.ds(16 * i, 16)] = jnp.zeros((16,), jnp.int32)
        cm_v[pl.ds(80, 16)] = negv
        gm_v[pl.ds(48, 16)] = negv

    def per_pair(pi, _):
        q0 = wid * nper + 2 * pi
        qs = (q0, q0 + 1)
        d0 = pltpu.async_copy(m_hbm.at[:, qs[0]], b0[0], sem0)
        d1 = pltpu.async_copy(m_hbm.at[:, qs[1]], b1[0], sem1)
        d0.wait()
        d1.wait()
        for j in range(NCH // 16):
            jv = lax.broadcast(jnp.int32(j), (16,))
            for (m_v, gm_v, *_r) in (b0, b1):
                plsc.store_scatter(gm_v, [jv], _splat_f(jnp.max(m_v[j, :])),
                                   mask=lane0)

        def sel_body(t, _c):
            tv = lax.broadcast(t, (16,))
            for (m_v, gm_v, cidx_v, _a, cm_v, *_r) in (b0, b1):
                g = jnp.full((16,), NEG, jnp.float32)
                for i in range(4):
                    g = jnp.maximum(g, gm_v[pl.ds(16 * i, 16)])
                s = jnp.max(g)
                jst = big
                for i in range(4):
                    v = gm_v[pl.ds(16 * i, 16)]
                    jst = jnp.minimum(jst, jnp.min(
                        jnp.where(v == s, iota + 16 * i, big)))
                jv = lax.broadcast(jst, (16,))
                mrow = plsc.load_gather(m_v, [jv, iota])
                lst = jnp.min(jnp.where(mrow == s, iota, big))
                cst = jst * 16 + lst
                plsc.store_scatter(cidx_v, [tv], lax.broadcast(cst, (16,)),
                                   mask=lane0)
                plsc.store_scatter(cm_v, [tv], _splat_f(s), mask=lane0)
                plsc.store_scatter(m_v, [jv, lax.broadcast(lst, (16,))],
                                   negv, mask=lane0)
                mrow2 = plsc.load_gather(m_v, [jv, iota])
                plsc.store_scatter(gm_v, [jv], _splat_f(jnp.max(mrow2)),
                                   mask=lane0)
            return 0

        lax.fori_loop(0, COARSE, sel_body, 0)

        for i in range(5):
            b0[3][pl.ds(16 * i, 16)] = b0[2][pl.ds(16 * i, 16)] + qs[0] * NCH
            b1[3][pl.ds(16 * i, 16)] = b1[2][pl.ds(16 * i, 16)] + qs[1] * NCH
        g0 = pltpu.async_copy(sim_hbm.at[b0[3]], b0[5], sem0)
        g1 = pltpu.async_copy(sim_hbm.at[b1[3]], b1[5], sem1)
        g0.wait()
        g1.wait()

        def ext_body(t, _c):
            tv = lax.broadcast(t, (16,))
            for (_m, _g, cidx_v, _a, cm_v, ch_v, cv_v, ci_v, ci80_v, _ck) \
                    in (b0, b1):
                s = jnp.float32(NEG)
                for i in range(6):
                    s = jnp.maximum(s, jnp.max(cm_v[pl.ds(16 * i, 16)]))
                pst = big
                for i in range(6):
                    v = cm_v[pl.ds(16 * i, 16)]
                    pst = jnp.minimum(pst, jnp.min(
                        jnp.where(v == s, iota + 16 * i, big)))
                pv = lax.broadcast(pst, (16,))
                lst = big
                for kk in range(8):
                    v = plsc.load_gather(ch_v, [pv, 16 * kk + iota])
                    lst = jnp.minimum(lst, jnp.min(
                        jnp.where(v == s, 16 * kk + iota, big)))
                cch = jnp.max(plsc.load_gather(cidx_v, [pv]))
                gi = cch * CHW + lst
                plsc.store_scatter(cv_v, [tv], _splat_f(s), mask=lane0)
                plsc.store_scatter(ci_v, [tv], lax.broadcast(gi, (16,)),
                                   mask=lane0)
                plsc.store_scatter(ci80_v, [tv], lax.broadcast(gi, (16,)),
                                   mask=lane0)
                plsc.store_scatter(ch_v, [pv, lax.broadcast(lst, (16,))],
                                   negv, mask=lane0)
                nm = jnp.full((16,), NEG, jnp.float32)
                for kk in range(8):
                    nm = jnp.maximum(nm, plsc.load_gather(
                        ch_v, [pv, 16 * kk + iota]))
                plsc.store_scatter(cm_v, [pv], _splat_f(jnp.max(nm)),
                                   mask=lane0)
            return 0

        lax.fori_loop(0, COARSE, ext_body, 0)

        k0 = pltpu.async_copy(keys_hbm.at[b0[8]], b0[9], sem0)
        k1 = pltpu.async_copy(keys_hbm.at[b1[8]], b1[9], sem1)
        k0.wait()
        k1.wait()
        for bb, q in ((b0, qs[0]), (b1, qs[1])):
            pltpu.sync_copy(bb[6], cv_hbm.at[q])
            pltpu.sync_copy(bb[7], ci_hbm.at[q])
            pltpu.sync_copy(bb[9], ck_hbm.at[q])
        return 0

    lax.fori_loop(0, nper // 2, per_pair, 0)


def _topk_call(m, sim2, keys_pad):
    mesh = plsc.VectorSubcoreMesh(core_axis_name="c", subcore_axis_name="s")
    f = pl.kernel(
        _topk_body,
        out_type=[
            jax.ShapeDtypeStruct((B, 128), jnp.float32),
            jax.ShapeDtypeStruct((B, 128), jnp.int32),
            jax.ShapeDtypeStruct((B, COARSE, 128), jnp.float32),
        ],
        mesh=mesh,
        compiler_params=pltpu.CompilerParams(needs_layout_passes=False),
        scratch_types=[
            pltpu.VMEM((NCH // 16, 16), jnp.float32),
            pltpu.VMEM((64,), jnp.float32),
            pltpu.VMEM((COARSE,), jnp.int32),
            pltpu.VMEM((COARSE,), jnp.int32),
            pltpu.VMEM((96,), jnp.float32),
            pltpu.VMEM((COARSE, CHW), jnp.float32),
            pltpu.VMEM((128,), jnp.float32),
            pltpu.VMEM((128,), jnp.int32),
            pltpu.VMEM((COARSE,), jnp.int32),
            pltpu.VMEM((COARSE, 128), jnp.float32),
        ] * 2 + [
            pltpu.SemaphoreType.DMA,
            pltpu.SemaphoreType.DMA,
        ],
    )
    return f(m, sim2, keys_pad)


def _gate_body(qn_ref, ck_ref, cv_ref, cif_ref,
               wq_ref, wk_ref, wcv_ref, wrk_ref, b1_ref, w2_ref, b2_ref,
               wf_ref, fif_ref):
    ckf = ck_ref[...][:, :CTX]
    nrm = jnp.sqrt(jnp.sum(ckf * ckf, axis=1, keepdims=True))
    ckn = ckf / (nrm + 1e-8)
    hk = _bdot(ckn, wk_ref[...])
    hq = _bdot(qn_ref[...], wq_ref[...])
    h3 = hk.reshape(QB_D, COARSE, GH) + hq[:, None, :]
    cv = cv_ref[...][:, :COARSE]
    h3 = h3 + _b(cv)[..., None] * _b(wcv_ref[...])[None, :, :]
    rank = lax.broadcasted_iota(jnp.int32, (QB_D, COARSE, 1), 1).astype(
        jnp.float32) / COARSE
    h3 = h3 + _b(rank) * _b(wrk_ref[...])[None, :, :]
    h3 = h3 + b1_ref[...][None, :, :]
    g3 = _gelu(h3)
    gl = jnp.sum(_b(g3) * _b(w2_ref[...])[None, :, :], axis=2) + b2_ref[0, 0]
    gate = jax.nn.sigmoid(gl)
    fused = ALPHA * cv + (1.0 - ALPHA) * gate
    cif = cif_ref[...][:, :COARSE]
    lane = lax.broadcasted_iota(jnp.int32, (QB_D, COARSE), 1)
    cur = fused
    fvals = []
    fidx = []
    for t in range(TOPM):
        a = jnp.argmax(cur, axis=1)
        oh = lane == a[:, None]
        fvals.append(jnp.max(cur, axis=1))
        fidx.append(jnp.sum(jnp.where(oh, cif, 0.0), axis=1))
        cur = jnp.where(oh, NEG, cur)
    fv = jnp.stack(fvals, axis=1)
    fi = jnp.stack(fidx, axis=1)
    e = jnp.exp((fv - fv[:, :1]) / TEMP)
    w = e / jnp.sum(e, axis=1, keepdims=True)
    wf_ref[...] = jnp.concatenate(
        [w, jnp.zeros((QB_D, 128 - TOPM), jnp.float32)], axis=1)
    fif_ref[...] = jnp.concatenate(
        [fi, jnp.broadcast_to(fi[:, :1], (QB_D, 128 - TOPM))], axis=1)


def _gate_call(qn, ckflat, cv, cif, wq, wk, wcv, wrk, b1, w2t, b2):
    nq = B // QB_D
    full = lambda shp: pl.BlockSpec(shp, lambda i: (0, 0))
    return pl.pallas_call(
        _gate_body,
        grid=(nq,),
        in_specs=[
            pl.BlockSpec((QB_D, CTX), lambda i: (i, 0)),
            pl.BlockSpec((QB_D * COARSE, 128), lambda i: (i, 0)),
            pl.BlockSpec((QB_D, 128), lambda i: (i, 0)),
            pl.BlockSpec((QB_D, 128), lambda i: (i, 0)),
            full((CTX, GH)), full((CTX, GH)), full((1, GH)), full((1, GH)),
            full((1, GH)), full((1, GH)), full((1, 1)),
        ],
        out_specs=[
            pl.BlockSpec((QB_D, 128), lambda i: (i, 0)),
            pl.BlockSpec((QB_D, 128), lambda i: (i, 0)),
        ],
        out_shape=[
            jax.ShapeDtypeStruct((B, 128), jnp.float32),
            jax.ShapeDtypeStruct((B, 128), jnp.float32),
        ],
    )(qn, ckflat, cv, cif, wq, wk, wcv, wrk, b1, w2t, b2)


def _series_body(series_hbm, fi_hbm, w_hbm, out_hbm,
                 fi0, w0, sv0, ob0, fi1, w1, sv1, ob1, sem0, sem1):
    wid = lax.axis_index("s") * 2 + lax.axis_index("c")
    nper = B // 32
    iota = lax.iota(jnp.int32, 16)
    sets = ((fi0, w0, sv0, ob0, sem0), (fi1, w1, sv1, ob1, sem1))

    def per_pair(pi, _):
        q0 = wid * nper + 2 * pi
        qs = (q0, q0 + 1)
        hdr = []
        for (fi_v, w_v, _s, _o, sem), q in zip(sets, qs):
            hdr.append(pltpu.async_copy(fi_hbm.at[q], fi_v, sem))
            hdr.append(pltpu.async_copy(w_hbm.at[q], w_v, sem))
        for d in hdr:
            d.wait()
        descs = []
        for (fi_v, _w, sv, _o, sem) in sets:
            fia = fi_v[pl.ds(0, 16)]
            fib = fi_v[pl.ds(16, 16)]
            for m in range(TOPM):
                fm = fia[m] if m < 16 else fib[m - 16]
                descs.append(pltpu.async_copy(series_hbm.at[fm],
                                              sv.at[m], sem))
        for d in descs:
            d.wait()
        for kk in range(SER // 16):
            def mb(m, accs):
                mv = lax.broadcast(m, (16,))
                a0 = accs[0] + plsc.load_gather(w0, [mv]) * \
                    plsc.load_gather(sv0, [mv, 16 * kk + iota])
                a1 = accs[1] + plsc.load_gather(w1, [mv]) * \
                    plsc.load_gather(sv1, [mv, 16 * kk + iota])
                return (a0, a1)
            z = jnp.zeros((16,), jnp.float32)
            r0, r1 = lax.fori_loop(0, TOPM, mb, (z, z))
            ob0[pl.ds(16 * kk, 16)] = r0
            ob1[pl.ds(16 * kk, 16)] = r1
        pltpu.sync_copy(ob0, out_hbm.at[qs[0]])
        pltpu.sync_copy(ob1, out_hbm.at[qs[1]])
        return 0

    lax.fori_loop(0, nper // 2, per_pair, 0)


def _series_call(series2, fi_sc, w_sc):
    mesh = plsc.VectorSubcoreMesh(core_axis_name="c", subcore_axis_name="s")
    f = pl.kernel(
        _series_body,
        out_type=jax.ShapeDtypeStruct((B, SER), jnp.float32),
        mesh=mesh,
        compiler_params=pltpu.CompilerParams(needs_layout_passes=False),
        scratch_types=[
            pltpu.VMEM((32,), jnp.int32),
            pltpu.VMEM((32,), jnp.float32),
            pltpu.VMEM((TOPM, SER), jnp.float32),
            pltpu.VMEM((SER,), jnp.float32),
        ] * 2 + [
            pltpu.SemaphoreType.DMA,
            pltpu.SemaphoreType.DMA,
        ],
    )
    return f(series2, fi_sc, w_sc)


EMB_ORDER = [
    ('emb_dataset', 'dataset_id', None),
    ('emb_sensor', 'sensor_type_id', None),
    ('emb_location', 'physical_location_id', None),
    ('emb_hour', 'hour', (0, 23)),
    ('emb_weekday', 'day_of_week', (0, 6)),
    ('emb_month', 'month', (1, 12)),
    ('emb_week', 'week_of_year', (1, 53)),
    ('emb_season', 'season_id', (0, 3)),
    ('emb_holiday', 'is_holiday', (0, 1)),
    ('emb_peak', 'peak_status_id', (0, 1)),
    ('emb_regime', 'regime_id', (0, 15)),
    ('emb_event', 'event_id', (0, 7)),
    ('emb_trend', 'trend_state_id', (0, 2)),
    ('emb_vol', 'volatility_state_id', (0, 2)),
    ('emb_shape', 'shape_state_id', (0, 2)),
    ('emb_rel', 'reliability_id', (0, 2)),
    ('emb_mend', 'month_end_flag', (0, 1)),
]


def kernel(dataset_id, sensor_type_id, physical_location_id, hour, day_of_week,
           month, week_of_year, season_id, is_holiday, peak_status_id,
           regime_id, event_id, trend_state_id, volatility_state_id,
           shape_state_id, reliability_id, month_end_flag,
           exogenous_vars, local_state_by_period, pool_keys, pool_series,
           emb_dataset, emb_sensor, emb_location, emb_hour, emb_weekday,
           emb_month, emb_week, emb_season, emb_holiday, emb_peak,
           emb_regime, emb_event, emb_trend, emb_vol, emb_shape, emb_rel,
           emb_mend, cat_w1, cat_b1, cat_w2, cat_b2, exo_w1, exo_b1,
           exo_w2, exo_b2, loc_w1, loc_b1, loc_w2, loc_b2, ln_g, ln_b,
           gate_w1, gate_b1, gate_w2, gate_b2):
    kw = dict(locals())
    parts = []
    for emb, idn, clip in EMB_ORDER:
        ids = kw[idn]
        if clip is not None:
            ids = jnp.clip(ids, clip[0], clip[1])
        parts.append(jnp.take(kw[emb], ids, axis=0))
    cat = jnp.concatenate(parts, axis=1)
    ls12 = local_state_by_period.reshape(B, 12)
    r1 = lambda a: a.reshape(1, -1)

    qn = _ctx_call(cat, exogenous_vars, ls12,
                   cat_w1, r1(cat_b1), cat_w2, r1(cat_b2),
                   exo_w1, r1(exo_b1), exo_w2, r1(exo_b2),
                   loc_w1, r1(loc_b1), loc_w2, r1(loc_b2),
                   r1(ln_g), r1(ln_b))

    keys_pad = jnp.pad(pool_keys, ((0, KP - K), (0, 128 - CTX)))
    sim, m = _sim_call(qn, keys_pad)
    sim2 = sim.reshape(B * NCH, CHW)

    cv, ci, ck = _topk_call(m, sim2, keys_pad)

    wq = gate_w1[:CTX]
    wk = gate_w1[CTX:2 * CTX]
    wcv = gate_w1[2 * CTX].reshape(1, GH)
    wrk = gate_w1[2 * CTX + 1].reshape(1, GH)
    w2t = gate_w2.reshape(1, GH)
    wf, fif = _gate_call(qn, ck.reshape(B * COARSE, 128), cv,
                         ci.astype(jnp.float32),
                         wq, wk, wcv, wrk, r1(gate_b1), w2t,
                         gate_b2.reshape(1, 1))

    fi_sc = fif[:, :32].astype(jnp.int32)
    w_sc = wf[:, :32]
    out = _series_call(pool_series.reshape(K, SER), fi_sc, w_sc)
    return out.reshape(B, PRED, CH)
